# baseline (speedup 1.0000x reference)
import jax
import jax.numpy as jnp
from jax.experimental import pallas as pl

_XROW = 128


def _ple_body(x_ref, a_ref, b_ref, lb_ref, ub_ref, o_ref):
    xv = x_ref[...]
    xb = jnp.broadcast_to(xv[None, :, None, :], (2,) + xv.shape[:1] + (8, 128))
    a = a_ref[...]
    b = b_ref[...]
    lb = lb_ref[...]
    ub = ub_ref[...]
    t = (xb - b) * a
    o_ref[...] = jnp.minimum(jnp.maximum(t, lb), ub)


def kernel(x, bins):
    n = x.shape[0]
    nb = bins.shape[0]
    rows = n // _XROW
    x3 = x.reshape(rows, _XROW)

    lo = bins
    hi = jnp.concatenate([bins[1:], jnp.array([-1.0], dtype=bins.dtype)])
    inv = 1.0 / (hi - lo)
    neg = jnp.float32(-3.0e38)
    lbv = jnp.where((jnp.arange(nb) == 0) | (jnp.arange(nb) == nb - 1), neg, 0.0)
    ubv = jnp.where(jnp.arange(nb) == nb - 1, 0.0, 1.0).astype(jnp.float32)

    def sub(v):
        return jnp.broadcast_to(
            v.astype(jnp.float32).reshape(2, 1, 8, 1), (2, 1, 8, _XROW))

    a4, b4, lb4, ub4 = sub(inv), sub(lo), sub(lbv), sub(ubv)

    blk = 800
    grid = (rows + blk - 1) // blk
    rep = lambda i: (0, 0, 0, 0)
    out = pl.pallas_call(
        _ple_body,
        grid=(grid,),
        in_specs=[
            pl.BlockSpec((blk, _XROW), lambda i: (i, 0)),
            pl.BlockSpec((2, 1, 8, _XROW), rep),
            pl.BlockSpec((2, 1, 8, _XROW), rep),
            pl.BlockSpec((2, 1, 8, _XROW), rep),
            pl.BlockSpec((2, 1, 8, _XROW), rep),
        ],
        out_specs=pl.BlockSpec((2, blk, 8, _XROW), lambda i: (0, i, 0, 0)),
        out_shape=jax.ShapeDtypeStruct((2, rows, 8, _XROW), jnp.float32),
    )(x3, a4, b4, lb4, ub4)
    return out.transpose(1, 3, 0, 2).reshape(n, nb)[:, None, :]


# speedup vs baseline: 1.0043x; 1.0043x over previous
import jax
import jax.numpy as jnp
from jax.experimental import pallas as pl

_XROW = 128


def _ple_body(x_ref, a_ref, b_ref, lb_ref, ub_ref, o_ref):
    xv = x_ref[...]
    xb = jnp.broadcast_to(xv[None, :, None, :], (2,) + xv.shape[:1] + (8, 128))
    a = a_ref[...]
    b = b_ref[...]
    lb = lb_ref[...]
    ub = ub_ref[...]
    t = (xb - b) * a
    o_ref[...] = jnp.minimum(jnp.maximum(t, lb), ub)


def kernel(x, bins):
    n = x.shape[0]
    nb = bins.shape[0]
    rows = n // _XROW
    x3 = x.reshape(rows, _XROW)

    lo = bins
    hi = jnp.concatenate([bins[1:], jnp.array([-1.0], dtype=bins.dtype)])
    inv = 1.0 / (hi - lo)
    neg = jnp.float32(-3.0e38)
    lbv = jnp.where((jnp.arange(nb) == 0) | (jnp.arange(nb) == nb - 1), neg, 0.0)
    ubv = jnp.where(jnp.arange(nb) == nb - 1, 0.0, 1.0).astype(jnp.float32)

    def sub(v):
        return jnp.broadcast_to(
            v.astype(jnp.float32).reshape(2, 1, 8, 1), (2, 1, 8, _XROW))

    a4, b4, lb4, ub4 = sub(inv), sub(lo), sub(lbv), sub(ubv)

    blk = 1000
    grid = (rows + blk - 1) // blk
    rep = lambda i: (0, 0, 0, 0)
    out = pl.pallas_call(
        _ple_body,
        grid=(grid,),
        in_specs=[
            pl.BlockSpec((blk, _XROW), lambda i: (i, 0)),
            pl.BlockSpec((2, 1, 8, _XROW), rep),
            pl.BlockSpec((2, 1, 8, _XROW), rep),
            pl.BlockSpec((2, 1, 8, _XROW), rep),
            pl.BlockSpec((2, 1, 8, _XROW), rep),
        ],
        out_specs=pl.BlockSpec((2, blk, 8, _XROW), lambda i: (0, i, 0, 0)),
        out_shape=jax.ShapeDtypeStruct((2, rows, 8, _XROW), jnp.float32),
    )(x3, a4, b4, lb4, ub4)
    return out.transpose(1, 3, 0, 2).reshape(n, nb)[:, None, :]
